# Initial kernel scaffold; baseline (speedup 1.0000x reference)
import jax
import jax.numpy as jnp
from jax.experimental import pallas as pl


def kernel(x, W, b):
    raise NotImplementedError("write your pallas kernel here")


# speedup vs baseline: 1.4088x; 1.4088x over previous
import jax
import jax.numpy as jnp
from jax.experimental import pallas as pl
from jax.experimental.pallas import tpu as pltpu

S = 16384
DIM = 2048
E = 16
K = 2
TS = 1024


def _gate_body(x_ref, w_ref, b_ref, idx_ref, val_ref, aux_ref,
               imp_acc, load_acc):
    step = pl.program_id(0)
    nsteps = pl.num_programs(0)

    logits = jnp.dot(x_ref[...], w_ref[...],
                     preferred_element_type=jnp.float32) + b_ref[...]
    m = jnp.max(logits, axis=-1, keepdims=True)
    ex = jnp.exp(logits - m)
    denom = jnp.sum(ex, axis=-1, keepdims=True)
    probs = ex / denom

    lane = jax.lax.broadcasted_iota(jnp.int32, (TS, E), 1)

    v1 = jnp.max(probs, axis=-1, keepdims=True)
    i1 = jnp.min(jnp.where(probs == v1, lane, E), axis=-1, keepdims=True)
    masked = jnp.where(lane == i1, -jnp.inf, probs)
    v2 = jnp.max(masked, axis=-1, keepdims=True)
    i2 = jnp.min(jnp.where(masked == v2, lane, E), axis=-1, keepdims=True)

    idx_ref[...] = jnp.concatenate([i1, i2], axis=1)
    val_ref[...] = jnp.concatenate([v1, v2], axis=1)

    imp_part = jnp.sum(probs, axis=0, keepdims=True)
    one_hot = (i1 == lane).astype(jnp.float32)
    load_part = jnp.sum(one_hot, axis=0, keepdims=True)

    @pl.when(step == 0)
    def _init():
        imp_acc[...] = imp_part
        load_acc[...] = load_part

    @pl.when(step != 0)
    def _accum():
        imp_acc[...] += imp_part
        load_acc[...] += load_part

    @pl.when(step == nsteps - 1)
    def _finalize():
        imp = imp_acc[...] / jnp.float32(S)
        load = load_acc[...] / jnp.float32(S)
        aux_ref[...] = jnp.float32(E) * jnp.sum(imp * load, axis=-1,
                                                keepdims=True)


def kernel(x, W, b):
    b2 = b.reshape(1, E)
    grid = (S // TS,)
    out_shapes = (
        jax.ShapeDtypeStruct((S, K), jnp.int32),
        jax.ShapeDtypeStruct((S, K), jnp.float32),
        jax.ShapeDtypeStruct((1, 1), jnp.float32),
    )
    topk_idx, topk_vals, aux = pl.pallas_call(
        _gate_body,
        grid=grid,
        in_specs=[
            pl.BlockSpec((TS, DIM), lambda i: (i, 0)),
            pl.BlockSpec((DIM, E), lambda i: (0, 0)),
            pl.BlockSpec((1, E), lambda i: (0, 0)),
        ],
        out_specs=(
            pl.BlockSpec((TS, K), lambda i: (i, 0)),
            pl.BlockSpec((TS, K), lambda i: (i, 0)),
            pl.BlockSpec((1, 1), lambda i: (0, 0)),
        ),
        out_shape=out_shapes,
        scratch_shapes=[
            pltpu.VMEM((1, E), jnp.float32),
            pltpu.VMEM((1, E), jnp.float32),
        ],
        compiler_params=pltpu.CompilerParams(
            dimension_semantics=("arbitrary",),
        ),
    )(x, W, b2)
    return (topk_idx, topk_vals, aux[0, 0])


# speedup vs baseline: 1.4533x; 1.0316x over previous
import jax
import jax.numpy as jnp
from jax.experimental import pallas as pl
from jax.experimental.pallas import tpu as pltpu

S = 16384
DIM = 2048
E = 16
K = 2
TS = 2048


def _gate_body(x_ref, w_ref, b_ref, idx_ref, val_ref, aux_ref,
               imp_acc, load_acc):
    step = pl.program_id(0)
    nsteps = pl.num_programs(0)

    logits = jnp.dot(x_ref[...], w_ref[...],
                     preferred_element_type=jnp.float32) + b_ref[...]
    m = jnp.max(logits, axis=-1, keepdims=True)
    ex = jnp.exp(logits - m)
    denom = jnp.sum(ex, axis=-1, keepdims=True)
    probs = ex / denom

    lane = jax.lax.broadcasted_iota(jnp.int32, (TS, E), 1)

    v1 = jnp.max(probs, axis=-1, keepdims=True)
    i1 = jnp.min(jnp.where(probs == v1, lane, E), axis=-1, keepdims=True)
    masked = jnp.where(lane == i1, -jnp.inf, probs)
    v2 = jnp.max(masked, axis=-1, keepdims=True)
    i2 = jnp.min(jnp.where(masked == v2, lane, E), axis=-1, keepdims=True)

    idx_ref[...] = jnp.concatenate([i1, i2], axis=1)
    val_ref[...] = jnp.concatenate([v1, v2], axis=1)

    imp_part = jnp.sum(probs, axis=0, keepdims=True)
    one_hot = (i1 == lane).astype(jnp.float32)
    load_part = jnp.sum(one_hot, axis=0, keepdims=True)

    @pl.when(step == 0)
    def _init():
        imp_acc[...] = imp_part
        load_acc[...] = load_part

    @pl.when(step != 0)
    def _accum():
        imp_acc[...] += imp_part
        load_acc[...] += load_part

    @pl.when(step == nsteps - 1)
    def _finalize():
        imp = imp_acc[...] / jnp.float32(S)
        load = load_acc[...] / jnp.float32(S)
        aux_ref[...] = jnp.float32(E) * jnp.sum(imp * load, axis=-1,
                                                keepdims=True)


def kernel(x, W, b):
    b2 = b.reshape(1, E)
    grid = (S // TS,)
    out_shapes = (
        jax.ShapeDtypeStruct((S, K), jnp.int32),
        jax.ShapeDtypeStruct((S, K), jnp.float32),
        jax.ShapeDtypeStruct((1, 1), jnp.float32),
    )
    topk_idx, topk_vals, aux = pl.pallas_call(
        _gate_body,
        grid=grid,
        in_specs=[
            pl.BlockSpec((TS, DIM), lambda i: (i, 0)),
            pl.BlockSpec((DIM, E), lambda i: (0, 0)),
            pl.BlockSpec((1, E), lambda i: (0, 0)),
        ],
        out_specs=(
            pl.BlockSpec((TS, K), lambda i: (i, 0)),
            pl.BlockSpec((TS, K), lambda i: (i, 0)),
            pl.BlockSpec((1, 1), lambda i: (0, 0)),
        ),
        out_shape=out_shapes,
        scratch_shapes=[
            pltpu.VMEM((1, E), jnp.float32),
            pltpu.VMEM((1, E), jnp.float32),
        ],
        compiler_params=pltpu.CompilerParams(
            dimension_semantics=("arbitrary",),
        ),
    )(x, W, b2)
    return (topk_idx, topk_vals, aux[0, 0])


# speedup vs baseline: 1.4555x; 1.0015x over previous
import jax
import jax.numpy as jnp
from jax.experimental import pallas as pl
from jax.experimental.pallas import tpu as pltpu

S = 16384
DIM = 2048
E = 16
K = 2
TS = 2048
DH = DIM // 2


def _gate_body(xa_ref, xb_ref, w_ref, b_ref, idx_ref, val_ref, aux_ref,
               imp_acc, load_acc):
    step = pl.program_id(0)
    nsteps = pl.num_programs(0)

    logits = (jnp.dot(xa_ref[...], w_ref[:DH, :],
                      preferred_element_type=jnp.float32)
              + jnp.dot(xb_ref[...], w_ref[DH:, :],
                        preferred_element_type=jnp.float32)
              + b_ref[...])
    m = jnp.max(logits, axis=-1, keepdims=True)
    ex = jnp.exp(logits - m)
    denom = jnp.sum(ex, axis=-1, keepdims=True)
    probs = ex / denom

    lane = jax.lax.broadcasted_iota(jnp.int32, (TS, E), 1)

    v1 = jnp.max(probs, axis=-1, keepdims=True)
    i1 = jnp.min(jnp.where(probs == v1, lane, E), axis=-1, keepdims=True)
    masked = jnp.where(lane == i1, -jnp.inf, probs)
    v2 = jnp.max(masked, axis=-1, keepdims=True)
    i2 = jnp.min(jnp.where(masked == v2, lane, E), axis=-1, keepdims=True)

    idx_ref[...] = jnp.concatenate([i1, i2], axis=1)
    val_ref[...] = jnp.concatenate([v1, v2], axis=1)

    imp_part = jnp.sum(probs, axis=0, keepdims=True)
    one_hot = (i1 == lane).astype(jnp.float32)
    load_part = jnp.sum(one_hot, axis=0, keepdims=True)

    @pl.when(step == 0)
    def _init():
        imp_acc[...] = imp_part
        load_acc[...] = load_part

    @pl.when(step != 0)
    def _accum():
        imp_acc[...] += imp_part
        load_acc[...] += load_part

    @pl.when(step == nsteps - 1)
    def _finalize():
        imp = imp_acc[...] / jnp.float32(S)
        load = load_acc[...] / jnp.float32(S)
        aux_ref[...] = jnp.float32(E) * jnp.sum(imp * load, axis=-1,
                                                keepdims=True)


def kernel(x, W, b):
    b2 = b.reshape(1, E)
    grid = (S // TS,)
    out_shapes = (
        jax.ShapeDtypeStruct((S, K), jnp.int32),
        jax.ShapeDtypeStruct((S, K), jnp.float32),
        jax.ShapeDtypeStruct((1, 1), jnp.float32),
    )
    topk_idx, topk_vals, aux = pl.pallas_call(
        _gate_body,
        grid=grid,
        in_specs=[
            pl.BlockSpec((TS, DH), lambda i: (i, 0)),
            pl.BlockSpec((TS, DH), lambda i: (i, 1)),
            pl.BlockSpec((DIM, E), lambda i: (0, 0)),
            pl.BlockSpec((1, E), lambda i: (0, 0)),
        ],
        out_specs=(
            pl.BlockSpec((TS, K), lambda i: (i, 0)),
            pl.BlockSpec((TS, K), lambda i: (i, 0)),
            pl.BlockSpec((1, 1), lambda i: (0, 0)),
        ),
        out_shape=out_shapes,
        scratch_shapes=[
            pltpu.VMEM((1, E), jnp.float32),
            pltpu.VMEM((1, E), jnp.float32),
        ],
        compiler_params=pltpu.CompilerParams(
            dimension_semantics=("arbitrary",),
        ),
    )(x, x, W, b2)
    return (topk_idx, topk_vals, aux[0, 0])
